# baseline (speedup 1.0000x reference)
import functools

import jax
import jax.numpy as jnp
from jax.experimental import pallas as pl
from jax.experimental.pallas import tpu as pltpu

_F32 = jnp.float32
_BF16 = jnp.bfloat16
_INV_SQRT2 = 0.7071067811865476


def _prologue_kernel(g, w_p_ref, w_l_ref, w_s_ref, w_u_ref, s_sign_ref,
                     scale_ref, loc_ref, w1_ref, b1_ref,
                     wcat_ref, logdet0_ref):
    C = w_p_ref.shape[0]
    Ch = C // 2
    dt = w_p_ref.dtype
    row = jax.lax.broadcasted_iota(jnp.int32, (C, C), 0)
    col = jax.lax.broadcasted_iota(jnp.int32, (C, C), 1)
    eye = (row == col)
    w_s = w_s_ref[...]
    s_sign = s_sign_ref[...]
    d = s_sign * jnp.exp(w_s)
    L = jnp.where(row > col, w_l_ref[...], jnp.where(eye, 1.0, 0.0).astype(dt))
    U = jnp.where(row < col, w_u_ref[...],
                  jnp.where(eye, jnp.broadcast_to(d, (C, C)), 0.0).astype(dt))
    weight = jnp.dot(jnp.dot(w_p_ref[...], L, preferred_element_type=_F32),
                     U, preferred_element_type=_F32)
    scale = scale_ref[...]
    loc = loc_ref[...]
    w_eff = weight * scale
    b_eff = jnp.dot(weight, (scale * loc).reshape(C, 1),
                    preferred_element_type=_F32)
    w1 = w1_ref[...]
    w1h = w1[:, :C]
    w1b = w1[:, C:]
    m2 = jnp.dot(w1b, w_eff[Ch:], preferred_element_type=_F32)
    c2 = jnp.dot(w1b, b_eff[Ch:], preferred_element_type=_F32) + b1_ref[...]
    top = jnp.concatenate([jnp.zeros((C, C), dt), w_eff, b_eff], axis=1)
    bot = jnp.concatenate([w1h, m2, c2], axis=1) * _INV_SQRT2
    wcat_ref[...] = jnp.concatenate([top, bot], axis=0).astype(_BF16)
    ld0 = g * (jnp.sum(jnp.log(jnp.abs(scale))) + jnp.sum(w_s))
    logdet0_ref[...] = ld0.reshape(1, 1)


def _main_kernel(in_ref, his_ref, wcat_ref, w2_ref, b2_ref, logdet0_ref,
                 out_ref, ld_ref):
    BB, C, G = in_ref.shape
    Ch = C // 2
    H = w2_ref.shape[1]
    wcat = wcat_ref[...]
    w2 = w2_ref[...]
    b2 = b2_ref[...]
    ld0 = logdet0_ref[0, 0]
    ones = jnp.ones((1, G), dtype=_BF16)
    for i in range(BB):
        x = in_ref[i].astype(_BF16)
        he = his_ref[i].astype(_BF16)
        xha = jnp.concatenate([he, x, ones], axis=0)
        r = jnp.dot(wcat, xha, preferred_element_type=_F32)
        in_a = r[:Ch]
        in_b = r[Ch:C]
        u = r[C:].astype(_BF16)
        m = u * jnp.bfloat16(1.4142135623730951 * 0.5)
        gl = m + m * jax.lax.erf(u)
        h2 = jnp.dot(w2, gl, preferred_element_type=_F32).astype(_BF16) + b2
        s = jnp.tanh(h2 * jnp.bfloat16(0.5)) * jnp.bfloat16(0.5) + jnp.bfloat16(0.5)
        log_s = s[:Ch]
        t = s[Ch:]
        e = jnp.exp(log_s)
        out_ref[i, :Ch] = e.astype(_F32) * in_a + t.astype(_F32)
        out_ref[i, Ch:] = in_b
        ld_ref[i:i + 1] = (ld0 + jnp.sum(log_s.astype(_F32))).reshape(1, 1, 1)


@functools.partial(jax.jit, static_argnames=("interpret",))
def _flow(input, his_enc, loc, scale, w_p, w_l, w_s, w_u, s_sign, W1, b1, W2,
          b2, interpret=False):
    B, C, G = input.shape
    H = W1.shape[0]
    dt = input.dtype

    wcat, logdet0 = pl.pallas_call(
        functools.partial(_prologue_kernel, float(G)),
        out_shape=(
            jax.ShapeDtypeStruct((C + H, 2 * C + 1), _BF16),
            jax.ShapeDtypeStruct((1, 1), dt),
        ),
        interpret=interpret,
    )(w_p, w_l, w_s.reshape(1, C), w_u, s_sign.reshape(1, C),
      scale.reshape(1, C), loc.reshape(1, C), W1, b1.reshape(H, 1))

    BB = 8
    out, ld = pl.pallas_call(
        _main_kernel,
        grid=(B // BB,),
        in_specs=[
            pl.BlockSpec((BB, C, G), lambda b: (b, 0, 0)),
            pl.BlockSpec((BB, C, G), lambda b: (b, 0, 0)),
            pl.BlockSpec((C + H, 2 * C + 1), lambda b: (0, 0)),
            pl.BlockSpec((C, H), lambda b: (0, 0)),
            pl.BlockSpec((C, 1), lambda b: (0, 0)),
            pl.BlockSpec((1, 1), lambda b: (0, 0)),
        ],
        out_specs=(
            pl.BlockSpec((BB, C, G), lambda b: (b, 0, 0)),
            pl.BlockSpec((BB, 1, 1), lambda b: (b, 0, 0)),
        ),
        out_shape=(
            jax.ShapeDtypeStruct((B, C, G), dt),
            jax.ShapeDtypeStruct((B, 1, 1), dt),
        ),
        compiler_params=pltpu.CompilerParams(
            dimension_semantics=("parallel",),
            vmem_limit_bytes=128 * 1024 * 1024,
        ),
        interpret=interpret,
    )(input, his_enc, wcat, W2.astype(_BF16),
      b2.reshape(C, 1).astype(_BF16), logdet0)

    return out, ld.reshape(B)


def kernel(input, his_enc, loc, scale, w_p, w_l, w_s, w_u, s_sign, W1, b1, W2, b2):
    return _flow(input, his_enc, loc, scale, w_p, w_l, w_s, w_u, s_sign,
                 W1, b1, W2, b2)


# speedup vs baseline: 1.0200x; 1.0200x over previous
import functools

import jax
import jax.numpy as jnp
from jax.experimental import pallas as pl
from jax.experimental.pallas import tpu as pltpu

_F32 = jnp.float32
_BF16 = jnp.bfloat16
_INV_SQRT2 = 0.7071067811865476


def _prologue_kernel(g, w_p_ref, w_l_ref, w_s_ref, w_u_ref, s_sign_ref,
                     scale_ref, loc_ref, w1_ref, b1_ref,
                     wcat_ref, logdet0_ref):
    C = w_p_ref.shape[0]
    Ch = C // 2
    dt = w_p_ref.dtype
    row = jax.lax.broadcasted_iota(jnp.int32, (C, C), 0)
    col = jax.lax.broadcasted_iota(jnp.int32, (C, C), 1)
    eye = (row == col)
    w_s = w_s_ref[...]
    s_sign = s_sign_ref[...]
    d = s_sign * jnp.exp(w_s)
    L = jnp.where(row > col, w_l_ref[...], jnp.where(eye, 1.0, 0.0).astype(dt))
    U = jnp.where(row < col, w_u_ref[...],
                  jnp.where(eye, jnp.broadcast_to(d, (C, C)), 0.0).astype(dt))
    weight = jnp.dot(jnp.dot(w_p_ref[...], L, preferred_element_type=_F32),
                     U, preferred_element_type=_F32)
    scale = scale_ref[...]
    loc = loc_ref[...]
    w_eff = weight * scale
    b_eff = jnp.dot(weight, (scale * loc).reshape(C, 1),
                    preferred_element_type=_F32)
    w1 = w1_ref[...]
    w1h = w1[:, :C]
    w1b = w1[:, C:]
    m2 = jnp.dot(w1b, w_eff[Ch:], preferred_element_type=_F32)
    c2 = jnp.dot(w1b, b_eff[Ch:], preferred_element_type=_F32) + b1_ref[...]
    top = jnp.concatenate([jnp.zeros((C, C), dt), w_eff, b_eff], axis=1)
    bot = jnp.concatenate([w1h, m2, c2], axis=1) * _INV_SQRT2
    wcat_ref[...] = jnp.concatenate([top, bot], axis=0).astype(_BF16)
    ld0 = g * (jnp.sum(jnp.log(jnp.abs(scale))) + jnp.sum(w_s))
    logdet0_ref[...] = ld0.reshape(1, 1)


def _main_kernel(in_ref, his_ref, wcat_ref, w2_ref, b2_ref, logdet0_ref,
                 out_ref, ld_ref):
    BB, C, G = in_ref.shape
    Ch = C // 2
    H = w2_ref.shape[1]
    wcat = wcat_ref[...]
    w2 = w2_ref[...]
    b2 = b2_ref[...]
    ld0 = logdet0_ref[0, 0]
    ones = jnp.ones((1, G), dtype=_BF16)
    for i in range(BB):
        x = in_ref[i].astype(_BF16)
        he = his_ref[i].astype(_BF16)
        xha = jnp.concatenate([he, x, ones], axis=0)
        r = jnp.dot(wcat, xha, preferred_element_type=_F32)
        in_a = r[:Ch]
        in_b = r[Ch:C]
        u = r[C:].astype(_BF16)
        m = u * jnp.bfloat16(1.4142135623730951 * 0.5)
        gl = m + m * jax.lax.erf(u)
        h2 = jnp.dot(w2, gl, preferred_element_type=_F32).astype(_BF16) + b2
        s = jnp.tanh(h2 * jnp.bfloat16(0.5)) * jnp.bfloat16(0.5) + jnp.bfloat16(0.5)
        log_s = s[:Ch]
        t = s[Ch:]
        e = jnp.exp(log_s)
        out_ref[i, :Ch] = (e * in_a.astype(_BF16) + t).astype(_F32)
        out_ref[i, Ch:] = in_b
        ld_ref[i:i + 1] = (ld0 + jnp.sum(log_s.astype(_F32))).reshape(1, 1, 1)


@functools.partial(jax.jit, static_argnames=("interpret",))
def _flow(input, his_enc, loc, scale, w_p, w_l, w_s, w_u, s_sign, W1, b1, W2,
          b2, interpret=False):
    B, C, G = input.shape
    H = W1.shape[0]
    dt = input.dtype

    wcat, logdet0 = pl.pallas_call(
        functools.partial(_prologue_kernel, float(G)),
        out_shape=(
            jax.ShapeDtypeStruct((C + H, 2 * C + 1), _BF16),
            jax.ShapeDtypeStruct((1, 1), dt),
        ),
        interpret=interpret,
    )(w_p, w_l, w_s.reshape(1, C), w_u, s_sign.reshape(1, C),
      scale.reshape(1, C), loc.reshape(1, C), W1, b1.reshape(H, 1))

    BB = 16
    out, ld = pl.pallas_call(
        _main_kernel,
        grid=(B // BB,),
        in_specs=[
            pl.BlockSpec((BB, C, G), lambda b: (b, 0, 0)),
            pl.BlockSpec((BB, C, G), lambda b: (b, 0, 0)),
            pl.BlockSpec((C + H, 2 * C + 1), lambda b: (0, 0)),
            pl.BlockSpec((C, H), lambda b: (0, 0)),
            pl.BlockSpec((C, 1), lambda b: (0, 0)),
            pl.BlockSpec((1, 1), lambda b: (0, 0)),
        ],
        out_specs=(
            pl.BlockSpec((BB, C, G), lambda b: (b, 0, 0)),
            pl.BlockSpec((BB, 1, 1), lambda b: (b, 0, 0)),
        ),
        out_shape=(
            jax.ShapeDtypeStruct((B, C, G), dt),
            jax.ShapeDtypeStruct((B, 1, 1), dt),
        ),
        compiler_params=pltpu.CompilerParams(
            dimension_semantics=("parallel",),
            vmem_limit_bytes=128 * 1024 * 1024,
        ),
        interpret=interpret,
    )(input, his_enc, wcat, W2.astype(_BF16),
      b2.reshape(C, 1).astype(_BF16), logdet0)

    return out, ld.reshape(B)


def kernel(input, his_enc, loc, scale, w_p, w_l, w_s, w_u, s_sign, W1, b1, W2, b2):
    return _flow(input, his_enc, loc, scale, w_p, w_l, w_s, w_u, s_sign,
                 W1, b1, W2, b2)
